# baseline (speedup 1.0000x reference)
import functools

import jax
import jax.numpy as jnp
from jax import lax
from jax.experimental import pallas as pl
from jax.experimental.pallas import tpu as pltpu
from jax.experimental.pallas import tpu_sc as plsc

L = 1024
KNN = 30
KPAD = 32
NB = 4
N = NB * L
NC = 2
NS = 16
NW = NC * NS
E_PER_W = N // NW
HI = jax.lax.Precision.HIGHEST
BF = jnp.bfloat16
F32 = jnp.float32


def _knn_body(xf_ref, W1_ref, b1_ref, g1_ref, be1_ref, rm1_ref, rv1_ref,
              nbr_ref, h1_ref, vals_ref):
    b = pl.program_id(0)
    ft = xf_ref[0]
    sq = jnp.sum(ft * ft, axis=1, keepdims=True)
    sq_row = jnp.reshape(jnp.sum(ft * ft, axis=1), (1, L))
    G = jax.lax.dot_general(ft, ft, (((1,), (1,)), ((), ())))
    vals_ref[...] = sq + sq_row - 2.0 * G

    cols = jax.lax.broadcasted_iota(jnp.int32, (L, L), 1)
    tcols = jax.lax.broadcasted_iota(jnp.int32, (L, KPAD), 1)

    def step(t, nbr):
        vals = vals_ref[...]
        m = jnp.min(vals, axis=1, keepdims=True)
        eq = vals == m
        idxm = jnp.min(jnp.where(eq, cols, L), axis=1, keepdims=True)
        vals_ref[...] = jnp.where(cols == idxm, jnp.inf, vals)
        return jnp.where(tcols == t, idxm, nbr)

    nbr = jax.lax.fori_loop(
        0, KNN, step, jnp.zeros((L, KPAD), jnp.int32), unroll=2)
    nbr_ref[0] = nbr + b * L

    h = jax.lax.dot_general(ft, W1_ref[...], (((1,), (0,)), ((), ())))
    h = (h + b1_ref[0] - rm1_ref[0]) / jnp.sqrt(rv1_ref[0] + 1e-5) \
        * g1_ref[0] + be1_ref[0]
    h1_ref[0] = h


def _knn_call(xf, W1, b1, g1, be1, rm1, rv1):
    vec = lambda v: v.reshape(1, -1)
    full = lambda r: pl.BlockSpec((1, r.shape[1]), lambda i: (0, 0))
    hid = W1.shape[1]
    return pl.pallas_call(
        _knn_body,
        grid=(NB,),
        in_specs=[
            pl.BlockSpec((1, L, xf.shape[2]), lambda i: (i, 0, 0)),
            pl.BlockSpec(W1.shape, lambda i: (0, 0)),
            full(vec(b1)), full(vec(g1)), full(vec(be1)),
            full(vec(rm1)), full(vec(rv1)),
        ],
        out_specs=[
            pl.BlockSpec((1, L, KPAD), lambda i: (i, 0, 0)),
            pl.BlockSpec((1, L, hid), lambda i: (i, 0, 0)),
        ],
        out_shape=[
            jax.ShapeDtypeStruct((NB, L, KPAD), jnp.int32),
            jax.ShapeDtypeStruct((NB, L, hid), jnp.float32),
        ],
        scratch_shapes=[pltpu.VMEM((L, L), jnp.float32)],
    )(xf, W1, vec(b1), vec(g1), vec(be1), vec(rm1), vec(rv1))


GRP = 4
NIDX = GRP * KPAD
NGRP = E_PER_W // GRP


def _v2v_sc(d, with_deg):
    nj = d // 16
    mesh = plsc.VectorSubcoreMesh(core_axis_name="c", subcore_axis_name="s")
    rows_per_tile = N // NS

    out_type = [jax.ShapeDtypeStruct((NC, N, d), jnp.float32)]
    if with_deg:
        out_type.append(jax.ShapeDtypeStruct((NC, N), jnp.float32))

    scratch = [
        pltpu.VMEM((NGRP, NIDX), jnp.int32),
        pltpu.VMEM((2, NIDX, d), jnp.float32),
        pltpu.VMEM((2, NIDX, d), jnp.float32),
        pltpu.VMEM((NIDX,), jnp.float32),
        pltpu.VMEM((rows_per_tile,), jnp.float32),
        pltpu.VMEM_SHARED((N, d), jnp.float32),
        pltpu.VMEM_SHARED((N,), jnp.float32),
        pltpu.SemaphoreType.DMA,
        pltpu.SemaphoreType.DMA,
    ]

    def body(gnbr_flat_hbm, h_hbm, vsum_out, *rest):
        if with_deg:
            deg_out = rest[0]
            rest = rest[1:]
        (fidx, rows, rep, ones_v, zdeg, vsum_sh, deg_sh, gsem, ssem) = rest
        cid = lax.axis_index("c")
        sid = lax.axis_index("s")
        wid = sid * NC + cid
        base = wid * E_PER_W * KPAD

        zero16 = jnp.zeros((16,), F32)
        lane = lax.iota(jnp.int32, 16)
        def zrow(i, _):
            for j in range(nj):
                rows[0, i, pl.ds(16 * j, 16)] = zero16
            return 0
        lax.fori_loop(0, NIDX, zrow, 0)
        for half in range(rows_per_tile // NIDX):
            pltpu.sync_copy(
                rows.at[0],
                vsum_sh.at[pl.ds(sid * rows_per_tile + half * NIDX, NIDX)])
        if with_deg:
            for j in range(rows_per_tile // 16):
                zdeg[pl.ds(16 * j, 16)] = zero16
            pltpu.sync_copy(zdeg, deg_sh.at[pl.ds(sid * rows_per_tile,
                                                  rows_per_tile)])
            for c in range(NIDX // 16):
                v = jnp.where((lane + 16 * c) % KPAD < KNN, 1.0, 0.0)
                ones_v[pl.ds(16 * c, 16)] = v.astype(F32)
        for bb in range(2):
            for e in range(GRP):
                for r in range(KNN, KPAD):
                    for j in range(nj):
                        rep[bb, e * KPAD + r, pl.ds(16 * j, 16)] = zero16

        for g in range(NGRP):
            pltpu.sync_copy(gnbr_flat_hbm.at[pl.ds(base + g * NIDX, NIDX)],
                            fidx.at[g])
        plsc.subcore_barrier()

        def gather(g, bb):
            pltpu.async_copy(h_hbm.at[fidx.at[g]], rows.at[bb], gsem)

        def gather_wait(g, bb):
            pltpu.make_async_copy(h_hbm.at[fidx.at[g]], rows.at[bb],
                                  gsem).wait()

        def scatter(g, bb):
            pltpu.async_copy(rep.at[bb], vsum_sh.at[fidx.at[g]], ssem,
                             add=True)
            if with_deg:
                pltpu.async_copy(ones_v, deg_sh.at[fidx.at[g]], ssem,
                                 add=True)

        def scatter_wait(g, bb):
            pltpu.make_async_copy(rep.at[bb], vsum_sh.at[fidx.at[g]],
                                  ssem).wait()
            if with_deg:
                pltpu.make_async_copy(ones_v, deg_sh.at[fidx.at[g]],
                                      ssem).wait()

        gather(0, 0)
        gather(1, 1)

        def group(gh, _):
            for bb in range(2):
                g = gh * 2 + bb
                gather_wait(g, bb)

                @pl.when(g >= 2)
                def _():
                    scatter_wait(g - 2, bb)

                for e in range(GRP):
                    ebase = e * KPAD
                    for j in range(nj):
                        sl = pl.ds(16 * j, 16)
                        acc = rows[bb, ebase, sl]
                        for r in range(1, KNN):
                            acc = acc + rows[bb, ebase + r, sl]
                        acc = acc * (1.0 / KNN)
                        for r in range(KNN):
                            rep[bb, ebase + r, sl] = acc
                scatter(g, bb)

                @pl.when(g + 2 < NGRP)
                def _():
                    gather(g + 2, bb)
            return 0

        lax.fori_loop(0, NGRP // 2, group, 0, unroll=False)
        scatter_wait(NGRP - 2, 0)
        scatter_wait(NGRP - 1, 1)

        plsc.subcore_barrier()
        sl = pl.ds(sid * rows_per_tile, rows_per_tile)
        pltpu.sync_copy(vsum_sh.at[sl], vsum_out.at[cid, sl])
        if with_deg:
            pltpu.sync_copy(deg_sh.at[sl], deg_out.at[cid, sl])

    return pl.kernel(body, out_type=out_type, mesh=mesh,
                     scratch_types=scratch,
                     compiler_params=pltpu.CompilerParams(
                         use_tc_tiling_on_sc=False))


def _mid_body(v_ref, dp_ref, W2_ref, b2_ref, g2_ref, be2_ref, rm2_ref,
              rv2_ref, h2_ref, degc_ref):
    Vsum = v_ref[0] + v_ref[1]
    ones2 = jnp.ones((NC, 1), jnp.float32)
    deg = jax.lax.dot_general(dp_ref[...], ones2, (((0,), (0,)), ((), ())),
                              precision=HI)
    degc = jnp.maximum(deg, 1.0)
    degc_ref[...] = degc
    V = jax.nn.relu(Vsum / degc)
    h = jax.lax.dot_general(V, W2_ref[...], (((1,), (0,)), ((), ())))
    h2_ref[...] = (h + b2_ref[0] - rm2_ref[0]) / jnp.sqrt(rv2_ref[0] + 1e-5) \
        * g2_ref[0] + be2_ref[0]


def _mid_call(vp, degp, W2, b2, g2, be2, rm2, rv2):
    vec = lambda v: v.reshape(1, -1)
    nospec = lambda a: pl.BlockSpec(a.shape, lambda: tuple(0 for _ in a.shape))
    out_c = W2.shape[1]
    args = (vp, degp, W2, vec(b2), vec(g2), vec(be2), vec(rm2), vec(rv2))
    return pl.pallas_call(
        _mid_body,
        in_specs=[nospec(a) for a in args],
        out_specs=[
            pl.BlockSpec((N, out_c), lambda: (0, 0)),
            pl.BlockSpec((N, 1), lambda: (0, 0)),
        ],
        out_shape=[
            jax.ShapeDtypeStruct((N, out_c), jnp.float32),
            jax.ShapeDtypeStruct((N, 1), jnp.float32),
        ],
    )(*args)


def _fin_body(v_ref, degc_ref, out_ref):
    out_ref[...] = (v_ref[0] + v_ref[1]) / degc_ref[...]


def _fin_call(vp, degc):
    nospec = lambda a: pl.BlockSpec(a.shape, lambda: tuple(0 for _ in a.shape))
    return pl.pallas_call(
        _fin_body,
        in_specs=[nospec(vp), nospec(degc)],
        out_specs=pl.BlockSpec(vp.shape[1:], lambda: (0, 0)),
        out_shape=jax.ShapeDtypeStruct(vp.shape[1:], jnp.float32),
    )(vp, degc)


def kernel(x, W1, b1, g1, be1, rm1, rv1, W2, b2, g2, be2, rm2, rv2):
    B, C, H, W = x.shape
    hid, out_c = W1.shape[1], W2.shape[1]
    xf = x.reshape(B, L, C)

    nbr, h1 = _knn_call(xf, W1, b1, g1, be1, rm1, rv1)
    gnbr = nbr.reshape(N * KPAD)
    h1f = h1.reshape(N, hid)

    v1p, degp = _v2v_sc(hid, True)(gnbr, h1f)
    h2, degc = _mid_call(v1p, degp, W2, b2, g2, be2, rm2, rv2)
    v2p, = _v2v_sc(out_c, False)(gnbr, h2)
    out = _fin_call(v2p, degc)

    return out.reshape(B, -1, H, W)


# speedup vs baseline: 1.5205x; 1.5205x over previous
import jax
import jax.numpy as jnp
from jax.experimental import pallas as pl
from jax.experimental.pallas import tpu as pltpu

L = 1024
KNN = 30
BF = jnp.bfloat16
F32 = jnp.float32


def _body(xf_ref, W1_ref, b1_ref, g1_ref, be1_ref, rm1_ref, rv1_ref,
          W2_ref, b2_ref, g2_ref, be2_ref, rm2_ref, rv2_ref, out_ref,
          vals_ref):
    ft = xf_ref[0]
    sq = jnp.sum(ft * ft, axis=1, keepdims=True)
    sq_row = jnp.reshape(jnp.sum(ft * ft, axis=1), (1, L))
    G = jax.lax.dot_general(ft, ft, (((1,), (1,)), ((), ())))
    vals_ref[...] = sq + sq_row - 2.0 * G

    cols = jax.lax.broadcasted_iota(jnp.int32, (L, L), 1)
    tcols = jax.lax.broadcasted_iota(jnp.int32, (L, 32), 1)

    m0 = jnp.min(vals_ref[...], axis=1, keepdims=True)

    def step(t, carry):
        m, nbr = carry
        vals = vals_ref[...]
        eq = vals == m
        idxm = jnp.min(jnp.where(eq, cols, L), axis=1, keepdims=True)
        newvals = jnp.where(cols == idxm, jnp.inf, vals)
        vals_ref[...] = newvals
        m = jnp.min(newvals, axis=1, keepdims=True)
        return m, jnp.where(tcols == t, idxm, nbr)

    _, nbr = jax.lax.fori_loop(
        0, KNN, step, (m0, jnp.zeros((L, 32), jnp.int32)), unroll=2)

    M = jnp.zeros((L, L), jnp.float32)
    for t in range(KNN):
        M = M + (cols == nbr[:, t:t + 1]).astype(jnp.float32)

    Mb = M.astype(BF)
    ones_col = jnp.ones((L, 1), BF)
    deg = jax.lax.dot_general(Mb, ones_col, (((0,), (0,)), ((), ())),
                              preferred_element_type=F32)
    degc = jnp.maximum(deg, 1.0)

    def bn(h, g_r, be_r, rm_r, rv_r):
        return (h - rm_r[0]) / jnp.sqrt(rv_r[0] + 1e-5) * g_r[0] + be_r[0]

    def mdot(h, dims):
        h_hi = h.astype(BF)
        h_lo = (h - h_hi.astype(F32)).astype(BF)
        return (jax.lax.dot_general(Mb, h_hi, dims, preferred_element_type=F32)
                + jax.lax.dot_general(Mb, h_lo, dims, preferred_element_type=F32))

    def v2v(h):
        E = mdot(h, (((1,), (0,)), ((), ()))) * (1.0 / KNN)
        Vsum = mdot(E, (((0,), (0,)), ((), ())))
        return Vsum / degc

    h = jax.lax.dot_general(ft, W1_ref[...], (((1,), (0,)), ((), ())))
    h = bn(h + b1_ref[0], g1_ref, be1_ref, rm1_ref, rv1_ref)
    h = jax.nn.relu(v2v(h))
    h = jax.lax.dot_general(h, W2_ref[...], (((1,), (0,)), ((), ())))
    h = bn(h + b2_ref[0], g2_ref, be2_ref, rm2_ref, rv2_ref)
    out_ref[0] = v2v(h)


def kernel(x, W1, b1, g1, be1, rm1, rv1, W2, b2, g2, be2, rm2, rv2):
    B, C, H, W = x.shape
    xf = x.reshape(B, L, C)
    vec = lambda v: v.reshape(1, -1)
    full = lambda r: pl.BlockSpec((1, r.shape[1]), lambda i: (0, 0))

    out = pl.pallas_call(
        _body,
        grid=(B,),
        in_specs=[
            pl.BlockSpec((1, L, C), lambda i: (i, 0, 0)),
            pl.BlockSpec(W1.shape, lambda i: (0, 0)),
            full(vec(b1)), full(vec(g1)), full(vec(be1)),
            full(vec(rm1)), full(vec(rv1)),
            pl.BlockSpec(W2.shape, lambda i: (0, 0)),
            full(vec(b2)), full(vec(g2)), full(vec(be2)),
            full(vec(rm2)), full(vec(rv2)),
        ],
        out_specs=pl.BlockSpec((1, L, W2.shape[1]), lambda i: (i, 0, 0)),
        out_shape=jax.ShapeDtypeStruct((B, L, W2.shape[1]), jnp.float32),
        scratch_shapes=[pltpu.VMEM((L, L), jnp.float32)],
    )(xf, W1, vec(b1), vec(g1), vec(be1), vec(rm1), vec(rv1),
      W2, vec(b2), vec(g2), vec(be2), vec(rm2), vec(rv2))

    return out.reshape(B, -1, H, W)


# speedup vs baseline: 1.5458x; 1.0166x over previous
import jax
import jax.numpy as jnp
from jax.experimental import pallas as pl
from jax.experimental.pallas import tpu as pltpu

L = 1024
KNN = 30
BF = jnp.bfloat16
F32 = jnp.float32


def _body(xf_ref, W1_ref, b1_ref, g1_ref, be1_ref, rm1_ref, rv1_ref,
          W2_ref, b2_ref, g2_ref, be2_ref, rm2_ref, rv2_ref, out_ref,
          vals_ref):
    ft = xf_ref[0]
    sq = jnp.sum(ft * ft, axis=1, keepdims=True)
    sq_row = jnp.reshape(jnp.sum(ft * ft, axis=1), (1, L))
    G = jax.lax.dot_general(ft, ft, (((1,), (1,)), ((), ())))
    vals_ref[...] = sq + sq_row - 2.0 * G

    cols = jax.lax.broadcasted_iota(jnp.int32, (1, L), 1)
    tcols = jax.lax.broadcasted_iota(jnp.int32, (1, 32), 1)

    def step(t, nbr):
        vals = vals_ref[...]
        m = jnp.min(vals, axis=1, keepdims=True)
        eq = vals == m
        idxm = jnp.min(jnp.where(eq, cols, L), axis=1, keepdims=True)
        vals_ref[...] = jnp.where(cols == idxm, jnp.inf, vals)
        return jnp.where(tcols == t, idxm, nbr)

    nbr = jax.lax.fori_loop(
        0, KNN, step, jnp.zeros((L, 32), jnp.int32), unroll=2)

    M = jnp.zeros((L, L), jnp.float32)
    for t in range(KNN):
        M = M + (cols == nbr[:, t:t + 1]).astype(jnp.float32)

    Mb = M.astype(BF)
    ones_col = jnp.ones((L, 1), BF)
    deg = jax.lax.dot_general(Mb, ones_col, (((0,), (0,)), ((), ())),
                              preferred_element_type=F32)
    degc = jnp.maximum(deg, 1.0)

    def bn(h, g_r, be_r, rm_r, rv_r):
        return (h - rm_r[0]) / jnp.sqrt(rv_r[0] + 1e-5) * g_r[0] + be_r[0]

    def mdot(h, dims):
        h_hi = h.astype(BF)
        h_lo = (h - h_hi.astype(F32)).astype(BF)
        return (jax.lax.dot_general(Mb, h_hi, dims, preferred_element_type=F32)
                + jax.lax.dot_general(Mb, h_lo, dims, preferred_element_type=F32))

    def v2v(h):
        E = mdot(h, (((1,), (0,)), ((), ()))) * (1.0 / KNN)
        Vsum = mdot(E, (((0,), (0,)), ((), ())))
        return Vsum / degc

    h = jax.lax.dot_general(ft, W1_ref[...], (((1,), (0,)), ((), ())))
    h = bn(h + b1_ref[0], g1_ref, be1_ref, rm1_ref, rv1_ref)
    h = jax.nn.relu(v2v(h))
    h = jax.lax.dot_general(h, W2_ref[...], (((1,), (0,)), ((), ())))
    h = bn(h + b2_ref[0], g2_ref, be2_ref, rm2_ref, rv2_ref)
    out_ref[0] = v2v(h)


def kernel(x, W1, b1, g1, be1, rm1, rv1, W2, b2, g2, be2, rm2, rv2):
    B, C, H, W = x.shape
    xf = x.reshape(B, L, C)
    vec = lambda v: v.reshape(1, -1)
    full = lambda r: pl.BlockSpec((1, r.shape[1]), lambda i: (0, 0))

    out = pl.pallas_call(
        _body,
        grid=(B,),
        in_specs=[
            pl.BlockSpec((1, L, C), lambda i: (i, 0, 0)),
            pl.BlockSpec(W1.shape, lambda i: (0, 0)),
            full(vec(b1)), full(vec(g1)), full(vec(be1)),
            full(vec(rm1)), full(vec(rv1)),
            pl.BlockSpec(W2.shape, lambda i: (0, 0)),
            full(vec(b2)), full(vec(g2)), full(vec(be2)),
            full(vec(rm2)), full(vec(rv2)),
        ],
        out_specs=pl.BlockSpec((1, L, W2.shape[1]), lambda i: (i, 0, 0)),
        out_shape=jax.ShapeDtypeStruct((B, L, W2.shape[1]), jnp.float32),
        scratch_shapes=[pltpu.VMEM((L, L), jnp.float32)],
    )(xf, W1, vec(b1), vec(g1), vec(be1), vec(rm1), vec(rv1),
      W2, vec(b2), vec(g2), vec(be2), vec(rm2), vec(rv2))

    return out.reshape(B, -1, H, W)


# speedup vs baseline: 1.6909x; 1.0939x over previous
import jax
import jax.numpy as jnp
from jax.experimental import pallas as pl
from jax.experimental.pallas import tpu as pltpu

L = 1024
KNN = 30
BF = jnp.bfloat16
F32 = jnp.float32


def _body(xf_ref, W1_ref, b1_ref, g1_ref, be1_ref, rm1_ref, rv1_ref,
          W2_ref, b2_ref, g2_ref, be2_ref, rm2_ref, rv2_ref, out_ref,
          vals_ref):
    ft = xf_ref[0]
    sq = jnp.sum(ft * ft, axis=1, keepdims=True)
    sq_row = jnp.reshape(jnp.sum(ft * ft, axis=1), (1, L))
    G = jax.lax.dot_general(ft, ft, (((1,), (1,)), ((), ())))
    vals_ref[...] = sq + sq_row - 2.0 * G

    cols = jax.lax.broadcasted_iota(jnp.int32, (1, L), 1)
    tcols = jax.lax.broadcasted_iota(jnp.int32, (1, 32), 1)

    def step(t, nbr):
        vals = vals_ref[...]
        m = jnp.min(vals, axis=1, keepdims=True)
        eq = vals == m
        idxm = jnp.min(jnp.where(eq, cols, L), axis=1, keepdims=True)
        vals_ref[...] = jnp.where(cols == idxm, jnp.inf, vals)
        return jnp.where(tcols == t, idxm, nbr)

    nbr = jax.lax.fori_loop(
        0, KNN, step, jnp.zeros((L, 32), jnp.int32), unroll=2)

    cols16 = jax.lax.broadcasted_iota(jnp.int16, (1, L), 1)
    nbr16 = nbr.astype(jnp.int16)
    one_bf = jnp.ones((), BF)
    zero_bf = jnp.zeros((), BF)
    Mb = jnp.zeros((L, L), BF)
    for t in range(KNN):
        Mb = Mb + jnp.where(cols16 == nbr16[:, t:t + 1], one_bf, zero_bf)
    ones_col = jnp.ones((L, 1), BF)
    deg = jax.lax.dot_general(Mb, ones_col, (((0,), (0,)), ((), ())),
                              preferred_element_type=F32)
    degc = jnp.maximum(deg, 1.0)

    def bn(h, g_r, be_r, rm_r, rv_r):
        return (h - rm_r[0]) / jnp.sqrt(rv_r[0] + 1e-5) * g_r[0] + be_r[0]

    def mdot(h, dims):
        h_hi = h.astype(BF)
        h_lo = (h - h_hi.astype(F32)).astype(BF)
        return (jax.lax.dot_general(Mb, h_hi, dims, preferred_element_type=F32)
                + jax.lax.dot_general(Mb, h_lo, dims, preferred_element_type=F32))

    def v2v(h):
        E = mdot(h, (((1,), (0,)), ((), ()))) * (1.0 / KNN)
        Vsum = mdot(E, (((0,), (0,)), ((), ())))
        return Vsum / degc

    h = jax.lax.dot_general(ft, W1_ref[...], (((1,), (0,)), ((), ())))
    h = bn(h + b1_ref[0], g1_ref, be1_ref, rm1_ref, rv1_ref)
    h = jax.nn.relu(v2v(h))
    h = jax.lax.dot_general(h, W2_ref[...], (((1,), (0,)), ((), ())))
    h = bn(h + b2_ref[0], g2_ref, be2_ref, rm2_ref, rv2_ref)
    out_ref[0] = v2v(h)


def kernel(x, W1, b1, g1, be1, rm1, rv1, W2, b2, g2, be2, rm2, rv2):
    B, C, H, W = x.shape
    xf = x.reshape(B, L, C)
    vec = lambda v: v.reshape(1, -1)
    full = lambda r: pl.BlockSpec((1, r.shape[1]), lambda i: (0, 0))

    out = pl.pallas_call(
        _body,
        grid=(B,),
        in_specs=[
            pl.BlockSpec((1, L, C), lambda i: (i, 0, 0)),
            pl.BlockSpec(W1.shape, lambda i: (0, 0)),
            full(vec(b1)), full(vec(g1)), full(vec(be1)),
            full(vec(rm1)), full(vec(rv1)),
            pl.BlockSpec(W2.shape, lambda i: (0, 0)),
            full(vec(b2)), full(vec(g2)), full(vec(be2)),
            full(vec(rm2)), full(vec(rv2)),
        ],
        out_specs=pl.BlockSpec((1, L, W2.shape[1]), lambda i: (i, 0, 0)),
        out_shape=jax.ShapeDtypeStruct((B, L, W2.shape[1]), jnp.float32),
        scratch_shapes=[pltpu.VMEM((L, L), jnp.float32)],
    )(xf, W1, vec(b1), vec(g1), vec(be1), vec(rm1), vec(rv1),
      W2, vec(b2), vec(g2), vec(be2), vec(rm2), vec(rv2))

    return out.reshape(B, -1, H, W)
